# baseline (speedup 1.0000x reference)
import functools

import jax
import jax.numpy as jnp
from jax import lax
from jax.experimental import pallas as pl
from jax.experimental.pallas import tpu as pltpu
from jax.experimental.pallas import tpu_sc as plsc

B = 1024
P = B * (B - 1) // 2
NW = 32
CHUNK = P // NW
VECS = CHUNK // 16
H1V = 512
H1 = H1V * 16
H2 = CHUNK - H1
TWO_B_M1 = 2 * B - 1
OT_N = 1040
MAGIC = 0x5F3759DF
INV_SQRT2 = 0.7071067811865476


def _pairs_body(x_hbm, x1_hbm, x2_hbm, x_v, ot_v, o1_v, o2_v, sem_x, sem_o):
    wid = lax.axis_index("s") * 2 + lax.axis_index("c")
    base = wid * CHUNK
    cp_x = pltpu.make_async_copy(x_hbm, x_v, sem_x)
    cp_x.start()
    lane = lax.iota(jnp.int32, 16)

    def ot_body(t, c):
        iv = t * 16 + lane
        ot_v[pl.ds(t * 16, 16)] = (iv * (TWO_B_M1 - iv)) >> 1
        return c

    lax.fori_loop(0, OT_N // 16, ot_body, 0, unroll=5)
    cp_x.wait()

    def body(t, carry):
        k, hd = carry
        r = plsc.bitcast(
            jnp.int32(MAGIC) - (plsc.bitcast(hd, jnp.int32) >> 1), jnp.float32
        )
        r = r * (jnp.float32(1.5) - (jnp.float32(0.5) * hd) * r * r)
        w = hd * r
        i_f = jnp.float32(B - 0.5) - jnp.float32(INV_SQRT2) * w
        i0 = i_f.astype(jnp.int32)
        i0 = jnp.where(plsc.load_gather(ot_v, [i0]) > k, i0 - 1, i0)
        i0 = jnp.where(plsc.load_gather(ot_v, [i0]) > k, i0 - 1, i0)
        i0 = jnp.where(plsc.load_gather(ot_v, [i0]) > k, i0 - 1, i0)
        i0 = jnp.where(plsc.load_gather(ot_v, [i0 + 1]) <= k, i0 + 1, i0)
        j = (k - plsc.load_gather(ot_v, [i0])) + (i0 + 1)
        o1_v[pl.ds(t * 16, 16)] = plsc.load_gather(x_v, [i0])
        o2_v[pl.ds(t * 16, 16)] = plsc.load_gather(x_v, [j])
        return k + 16, hd - jnp.float32(64.0)

    def carry_at(k0):
        k = k0 + lane
        return k, jnp.float32(2095104.5) - jnp.float32(4.0) * k.astype(jnp.float32)

    lax.fori_loop(0, H1V, body, carry_at(base), unroll=8)
    cp1a = pltpu.make_async_copy(
        o1_v.at[pl.ds(0, H1)], x1_hbm.at[pl.ds(base, H1)], sem_o
    )
    cp1b = pltpu.make_async_copy(
        o2_v.at[pl.ds(0, H1)], x2_hbm.at[pl.ds(base, H1)], sem_o
    )
    cp1a.start()
    cp1b.start()
    lax.fori_loop(H1V, VECS, body, carry_at(base + H1), unroll=8)
    cp2a = pltpu.make_async_copy(
        o1_v.at[pl.ds(H1, H2)], x1_hbm.at[pl.ds(base + H1, H2)], sem_o
    )
    cp2b = pltpu.make_async_copy(
        o2_v.at[pl.ds(H1, H2)], x2_hbm.at[pl.ds(base + H1, H2)], sem_o
    )
    cp2a.start()
    cp2b.start()
    cp1a.wait()
    cp1b.wait()
    cp2a.wait()
    cp2b.wait()


@functools.cache
def _build():
    return functools.partial(
        pl.kernel,
        out_type=(
            jax.ShapeDtypeStruct((P,), jnp.float32),
            jax.ShapeDtypeStruct((P,), jnp.float32),
        ),
        mesh=plsc.VectorSubcoreMesh(
            core_axis_name="c", subcore_axis_name="s", num_cores=2, num_subcores=16
        ),
        scratch_types=[
            pltpu.VMEM((B,), jnp.float32),
            pltpu.VMEM((OT_N,), jnp.int32),
            pltpu.VMEM((CHUNK,), jnp.float32),
            pltpu.VMEM((CHUNK,), jnp.float32),
            pltpu.SemaphoreType.DMA,
            pltpu.SemaphoreType.DMA,
        ],
        compiler_params=pltpu.CompilerParams(needs_layout_passes=False),
    )(_pairs_body)


def kernel(x):
    return _build()(x)


# speedup vs baseline: 1.9477x; 1.9477x over previous
import functools

import jax
import jax.numpy as jnp
from jax import lax
from jax.experimental import pallas as pl
from jax.experimental.pallas import tpu as pltpu
from jax.experimental.pallas import tpu_sc as plsc

B = 1024
P = B * (B - 1) // 2
NW = 32
CHUNK = P // NW
VECS = CHUNK // 16
H1V = 512
H1 = H1V * 16
H2 = CHUNK - H1
TWO_B_M1 = 2 * B - 1
OT_N = 1040
MAGIC = 0x5F3759DF
INV_SQRT2 = 0.7071067811865476


def _row_offset(i):
    return (i * (TWO_B_M1 - i)) >> 1


def _pairs_body(x_hbm, x1_hbm, x2_hbm, x_v, o1_v, o2_v, sem_x, sem_o):
    wid = lax.axis_index("s") * 2 + lax.axis_index("c")
    base = wid * CHUNK
    cp_x = pltpu.make_async_copy(x_hbm, x_v, sem_x)
    cp_x.start()
    lane = lax.iota(jnp.int32, 16)
    cp_x.wait()

    def body(t, carry):
        k, hd = carry
        r = plsc.bitcast(
            jnp.int32(MAGIC) - (plsc.bitcast(hd, jnp.int32) >> 1), jnp.float32
        )
        r = r * (jnp.float32(1.5) - (jnp.float32(0.5) * hd) * r * r)
        w = hd * r
        i_f = jnp.float32(B - 0.5) - jnp.float32(INV_SQRT2) * w
        i0 = i_f.astype(jnp.int32)
        i0 = jnp.where(_row_offset(i0) > k, i0 - 1, i0)
        i0 = jnp.where(_row_offset(i0) > k, i0 - 1, i0)
        i0 = jnp.where(_row_offset(i0) > k, i0 - 1, i0)
        i0 = jnp.where(_row_offset(i0 + 1) <= k, i0 + 1, i0)
        j = (k - _row_offset(i0)) + (i0 + 1)
        o1_v[pl.ds(t * 16, 16)] = plsc.load_gather(x_v, [i0])
        o2_v[pl.ds(t * 16, 16)] = plsc.load_gather(x_v, [j])
        return k + 16, hd - jnp.float32(64.0)

    def carry_at(k0):
        k = k0 + lane
        return k, jnp.float32(2095104.5) - jnp.float32(4.0) * k.astype(jnp.float32)

    lax.fori_loop(0, H1V, body, carry_at(base), unroll=8)
    cp1a = pltpu.make_async_copy(
        o1_v.at[pl.ds(0, H1)], x1_hbm.at[pl.ds(base, H1)], sem_o
    )
    cp1b = pltpu.make_async_copy(
        o2_v.at[pl.ds(0, H1)], x2_hbm.at[pl.ds(base, H1)], sem_o
    )
    cp1a.start()
    cp1b.start()
    lax.fori_loop(H1V, VECS, body, carry_at(base + H1), unroll=8)
    cp2a = pltpu.make_async_copy(
        o1_v.at[pl.ds(H1, H2)], x1_hbm.at[pl.ds(base + H1, H2)], sem_o
    )
    cp2b = pltpu.make_async_copy(
        o2_v.at[pl.ds(H1, H2)], x2_hbm.at[pl.ds(base + H1, H2)], sem_o
    )
    cp2a.start()
    cp2b.start()
    cp1a.wait()
    cp1b.wait()
    cp2a.wait()
    cp2b.wait()


@functools.cache
def _build():
    return functools.partial(
        pl.kernel,
        out_type=(
            jax.ShapeDtypeStruct((P,), jnp.float32),
            jax.ShapeDtypeStruct((P,), jnp.float32),
        ),
        mesh=plsc.VectorSubcoreMesh(
            core_axis_name="c", subcore_axis_name="s", num_cores=2, num_subcores=16
        ),
        scratch_types=[
            pltpu.VMEM((B,), jnp.float32),
            pltpu.VMEM((CHUNK,), jnp.float32),
            pltpu.VMEM((CHUNK,), jnp.float32),
            pltpu.SemaphoreType.DMA,
            pltpu.SemaphoreType.DMA,
        ],
        compiler_params=pltpu.CompilerParams(needs_layout_passes=False),
    )(_pairs_body)


def kernel(x):
    return _build()(x)


# speedup vs baseline: 2.1954x; 1.1272x over previous
import functools

import jax
import jax.numpy as jnp
from jax import lax
from jax.experimental import pallas as pl
from jax.experimental.pallas import tpu as pltpu
from jax.experimental.pallas import tpu_sc as plsc

B = 1024
P = B * (B - 1) // 2
NW = 32
CHUNK = P // NW
HALF = 8192
H2 = CHUNK - HALF
TWO_B_M1 = 2 * B - 1
GUARD = 16
BUFN = GUARD + CHUNK + 32
XPAD = 1040
MAGIC = 0x5F3759DF
INV_SQRT2 = 0.7071067811865476


def _row_offset(i):
    return (i * (TWO_B_M1 - i)) >> 1


def _row_of(kscal):
    k = jnp.full((16,), kscal, jnp.int32)
    hd = jnp.float32(2095104.5) - jnp.float32(4.0) * k.astype(jnp.float32)
    r = plsc.bitcast(
        jnp.int32(MAGIC) - (plsc.bitcast(hd, jnp.int32) >> 1), jnp.float32
    )
    r = r * (jnp.float32(1.5) - (jnp.float32(0.5) * hd) * r * r)
    i_f = jnp.float32(B - 0.5) - jnp.float32(INV_SQRT2) * (hd * r)
    i0 = i_f.astype(jnp.int32)
    i0 = jnp.where(_row_offset(i0) > k, i0 - 1, i0)
    i0 = jnp.where(_row_offset(i0) > k, i0 - 1, i0)
    i0 = jnp.where(_row_offset(i0) > k, i0 - 1, i0)
    i0 = jnp.where(_row_offset(i0 + 1) <= k, i0 + 1, i0)
    return jnp.max(i0)


def _pairs_body(x_hbm, x1_hbm, x2_hbm, x_v, o1_v, o2_v, sem_x, sem_o):
    wid = lax.axis_index("s") * 2 + lax.axis_index("c")
    base = wid * CHUNK
    cp_x = pltpu.make_async_copy(x_hbm, x_v.at[pl.ds(0, B)], sem_x)
    cp_x.start()
    cp_x.wait()

    def walk(F, C, ceil_start):
        i_init = _row_of(base + F)
        pos_init = _row_offset(i_init) - base

        def cond(st):
            return st[1] < C

        def rbody(st):
            i, pos = st
            lim = jnp.minimum(jnp.int32(B - 1) - i, C - pos)
            d = jnp.int32(F) - pos
            if ceil_start:
                off0 = jnp.maximum(0, ((d + 15) >> 4) << 4)
            else:
                off0 = jnp.maximum(0, (d >> 4) << 4)
            n = jnp.maximum(0, (lim - off0 + 15) >> 4)
            splat = plsc.load_gather(x_v, [jnp.full((16,), i, jnp.int32)])
            p0 = pos + off0 + GUARD
            j0 = i + 1 + off0

            def ibody(t, c):
                o1_v[pl.ds(p0 + t * 16, 16)] = splat
                o2_v[pl.ds(p0 + t * 16, 16)] = x_v[pl.ds(j0 + t * 16, 16)]
                return c

            lax.fori_loop(0, n, ibody, 0)
            return i + 1, pos + (jnp.int32(B - 1) - i)

        lax.while_loop(cond, rbody, (i_init, pos_init))

    walk(0, HALF + 16, False)
    cp1a = pltpu.make_async_copy(
        o1_v.at[pl.ds(GUARD, HALF)], x1_hbm.at[pl.ds(base, HALF)], sem_o
    )
    cp1b = pltpu.make_async_copy(
        o2_v.at[pl.ds(GUARD, HALF)], x2_hbm.at[pl.ds(base, HALF)], sem_o
    )
    cp1a.start()
    cp1b.start()
    walk(HALF, CHUNK, True)
    cp2a = pltpu.make_async_copy(
        o1_v.at[pl.ds(GUARD + HALF, H2)], x1_hbm.at[pl.ds(base + HALF, H2)], sem_o
    )
    cp2b = pltpu.make_async_copy(
        o2_v.at[pl.ds(GUARD + HALF, H2)], x2_hbm.at[pl.ds(base + HALF, H2)], sem_o
    )
    cp2a.start()
    cp2b.start()
    cp1a.wait()
    cp1b.wait()
    cp2a.wait()
    cp2b.wait()


@functools.cache
def _build():
    return functools.partial(
        pl.kernel,
        out_type=(
            jax.ShapeDtypeStruct((P,), jnp.float32),
            jax.ShapeDtypeStruct((P,), jnp.float32),
        ),
        mesh=plsc.VectorSubcoreMesh(
            core_axis_name="c", subcore_axis_name="s", num_cores=2, num_subcores=16
        ),
        scratch_types=[
            pltpu.VMEM((XPAD,), jnp.float32),
            pltpu.VMEM((BUFN,), jnp.float32),
            pltpu.VMEM((BUFN,), jnp.float32),
            pltpu.SemaphoreType.DMA,
            pltpu.SemaphoreType.DMA,
        ],
        compiler_params=pltpu.CompilerParams(needs_layout_passes=False),
    )(_pairs_body)


def kernel(x):
    return _build()(x)
